# baseline (speedup 1.0000x reference)
import functools

import jax
import jax.numpy as jnp
from jax import lax
from jax.experimental import pallas as pl
from jax.experimental.pallas import tpu as pltpu
from jax.experimental.pallas import tpu_sc as plsc

_NC = 2
_NS = 16
_NW = _NC * _NS
_CHUNK = 128


def _sc_gather_pair(ids_flat, elem_weight, hnet_weight):
    n = ids_flat.shape[0]
    emb = elem_weight.shape[1]
    per_w = n // _NW
    n_chunks = per_w // _CHUNK
    mesh = plsc.VectorSubcoreMesh(core_axis_name="c", subcore_axis_name="s")

    @functools.partial(
        pl.kernel,
        out_type=(
            jax.ShapeDtypeStruct((n, emb), jnp.float32),
            jax.ShapeDtypeStruct((n, emb), jnp.float32),
        ),
        mesh=mesh,
        scratch_types=[
            pltpu.VMEM((_CHUNK,), jnp.int32),
            pltpu.VMEM((_CHUNK, emb), jnp.float32),
            pltpu.VMEM((_CHUNK, emb), jnp.float32),
            pltpu.SemaphoreType.DMA,
            pltpu.SemaphoreType.DMA,
        ],
        compiler_params=pltpu.CompilerParams(use_tc_tiling_on_sc=True),
    )
    def sc_gather(ids_hbm, elem_hbm, hnet_hbm, out_e, out_h,
                  idx_v, erow_v, hrow_v, sem_e, sem_h):
        wid = lax.axis_index("s") * _NC + lax.axis_index("c")
        base = wid * per_w

        @pl.loop(0, n_chunks)
        def _(j):
            off = base + j * _CHUNK
            pltpu.sync_copy(ids_hbm.at[pl.ds(off, _CHUNK)], idx_v)
            cp_e = pltpu.async_copy(elem_hbm.at[idx_v], erow_v, sem_e)
            cp_h = pltpu.async_copy(hnet_hbm.at[idx_v], hrow_v, sem_h)
            cp_e.wait()
            cp_h.wait()
            pltpu.sync_copy(erow_v, out_e.at[pl.ds(off, _CHUNK)])
            pltpu.sync_copy(hrow_v, out_h.at[pl.ds(off, _CHUNK)])

    return sc_gather(ids_flat, elem_weight, hnet_weight)


def _tc_combine(hnet3, erow3, hrow3, lin_weight, tb=8, interpret=False):
    b, l, nhp = hnet3.shape
    emb = lin_weight.shape[0]

    def body(hnet_ref, e_ref, h_ref, lin_ref, out_ref):
        for t in range(tb):
            scal = lax.dot_general(
                hnet_ref[t], lin_ref[...],
                (((1,), (1,)), ((), ())),
                preferred_element_type=jnp.float32,
            )
            out_ref[t] = e_ref[t, :l, :] + h_ref[t, :l, :] * scal

    lp = erow3.shape[1]
    return pl.pallas_call(
        body,
        grid=(b // tb,),
        in_specs=[
            pl.BlockSpec((tb, l, nhp), lambda i: (i, 0, 0)),
            pl.BlockSpec((tb, lp, emb), lambda i: (i, 0, 0)),
            pl.BlockSpec((tb, lp, emb), lambda i: (i, 0, 0)),
            pl.BlockSpec((emb, nhp), lambda i: (0, 0)),
        ],
        out_specs=pl.BlockSpec((tb, l, emb), lambda i: (i, 0, 0)),
        out_shape=jax.ShapeDtypeStruct((b, l, emb), jnp.float32),
        interpret=interpret,
    )(hnet3, erow3, hrow3, lin_weight)


def kernel(input_ids, hnet_tensor, elem_weight, hnet_weight, lin_weight):
    b, l = input_ids.shape
    emb = elem_weight.shape[1]
    lp = l + (-l) % 8
    ids_pad = jnp.pad(input_ids, ((0, 0), (0, lp - l)))
    ids_flat = ids_pad.reshape(b * lp).astype(jnp.int32)
    erow, hrow = _sc_gather_pair(ids_flat, elem_weight, hnet_weight)
    erow3 = erow.reshape(b, lp, emb)
    hrow3 = hrow.reshape(b, lp, emb)
    return _tc_combine(hnet_tensor, erow3, hrow3, lin_weight)


# speedup vs baseline: 2.4884x; 2.4884x over previous
import functools

import jax
import jax.numpy as jnp
from jax import lax
from jax.experimental import pallas as pl
from jax.experimental.pallas import tpu as pltpu
from jax.experimental.pallas import tpu_sc as plsc

_NC = 2
_NS = 16
_NW = _NC * _NS
_CHUNK = 128


def _sc_gather_pair(ids_flat, elem_weight, hnet_weight):
    n = ids_flat.shape[0]
    emb = elem_weight.shape[1]
    per_w = n // _NW
    n_chunks = per_w // _CHUNK
    mesh = plsc.VectorSubcoreMesh(core_axis_name="c", subcore_axis_name="s")

    @functools.partial(
        pl.kernel,
        out_type=(
            jax.ShapeDtypeStruct((n, emb), jnp.float32),
            jax.ShapeDtypeStruct((n, emb), jnp.float32),
        ),
        mesh=mesh,
        scratch_types=[
            pltpu.VMEM((_CHUNK,), jnp.int32),
            pltpu.VMEM((_CHUNK, emb), jnp.float32),
            pltpu.VMEM((_CHUNK, emb), jnp.float32),
            pltpu.SemaphoreType.DMA,
            pltpu.SemaphoreType.DMA,
        ],
        compiler_params=pltpu.CompilerParams(use_tc_tiling_on_sc=True),
    )
    def sc_gather(ids_hbm, elem_hbm, hnet_hbm, out_e, out_h,
                  idx_v, erow_v, hrow_v, sem_e, sem_h):
        wid = lax.axis_index("s") * _NC + lax.axis_index("c")
        base = wid * per_w

        @pl.loop(0, n_chunks)
        def _(j):
            off = base + j * _CHUNK
            pltpu.sync_copy(ids_hbm.at[pl.ds(off, _CHUNK)], idx_v)
            cp_e = pltpu.async_copy(elem_hbm.at[idx_v], erow_v, sem_e)
            cp_h = pltpu.async_copy(hnet_hbm.at[idx_v], hrow_v, sem_h)
            cp_e.wait()
            cp_h.wait()
            pltpu.sync_copy(erow_v, out_e.at[pl.ds(off, _CHUNK)])
            pltpu.sync_copy(hrow_v, out_h.at[pl.ds(off, _CHUNK)])

    return sc_gather(ids_flat, elem_weight, hnet_weight)


def _tc_combine(hnet3, erow3, hrow3, lin_weight, tb=8, interpret=False):
    b, l, nhp = hnet3.shape
    emb = lin_weight.shape[0]

    def body(hnet_ref, e_ref, h_ref, lin_ref, out_ref):
        for t in range(tb):
            scal = lax.dot_general(
                hnet_ref[t], lin_ref[...],
                (((1,), (1,)), ((), ())),
                preferred_element_type=jnp.float32,
            )
            out_ref[t] = e_ref[t, :l, :] + h_ref[t, :l, :] * scal

    lp = erow3.shape[1]
    return pl.pallas_call(
        body,
        grid=(b // tb,),
        in_specs=[
            pl.BlockSpec((tb, l, nhp), lambda i: (i, 0, 0)),
            pl.BlockSpec((tb, lp, emb), lambda i: (i, 0, 0)),
            pl.BlockSpec((tb, lp, emb), lambda i: (i, 0, 0)),
            pl.BlockSpec((emb, nhp), lambda i: (0, 0)),
        ],
        out_specs=pl.BlockSpec((tb, l, emb), lambda i: (i, 0, 0)),
        out_shape=jax.ShapeDtypeStruct((b, l, emb), jnp.float32),
        interpret=interpret,
    )(hnet3, erow3, hrow3, lin_weight)


def kernel(input_ids, hnet_tensor, elem_weight, hnet_weight, lin_weight):
    b, l = input_ids.shape
    emb = elem_weight.shape[1]
    lp = l + (-l) % 8
    ids_pad = jnp.pad(input_ids, ((0, 0), (0, lp - l)), mode="edge")
    ids_flat = ids_pad.reshape(b * lp).astype(jnp.int32)
    erow, hrow = _sc_gather_pair(ids_flat, elem_weight, hnet_weight)
    erow3 = erow.reshape(b, lp, emb)
    hrow3 = hrow.reshape(b, lp, emb)
    return _tc_combine(hnet_tensor, erow3, hrow3, lin_weight)


# speedup vs baseline: 2.5944x; 1.0426x over previous
import functools

import jax
import jax.numpy as jnp
from jax import lax
from jax.experimental import pallas as pl
from jax.experimental.pallas import tpu as pltpu
from jax.experimental.pallas import tpu_sc as plsc

_NC = 2
_NS = 16
_NW = _NC * _NS
_CHUNK = 128


def _sc_gather_pair(ids_flat, elem_weight, hnet_weight):
    n = ids_flat.shape[0]
    emb = elem_weight.shape[1]
    per_w = n // _NW
    n_chunks = per_w // _CHUNK
    mesh = plsc.VectorSubcoreMesh(core_axis_name="c", subcore_axis_name="s")

    @functools.partial(
        pl.kernel,
        out_type=(
            jax.ShapeDtypeStruct((n, emb), jnp.float32),
            jax.ShapeDtypeStruct((n, emb), jnp.float32),
        ),
        mesh=mesh,
        scratch_types=[
            pltpu.VMEM((_CHUNK,), jnp.int32),
            pltpu.VMEM((_CHUNK, emb), jnp.float32),
            pltpu.VMEM((_CHUNK, emb), jnp.float32),
            pltpu.SemaphoreType.DMA,
            pltpu.SemaphoreType.DMA,
        ],
        compiler_params=pltpu.CompilerParams(use_tc_tiling_on_sc=True),
    )
    def sc_gather(ids_hbm, elem_hbm, hnet_hbm, out_e, out_h,
                  idx_v, erow_v, hrow_v, sem_e, sem_h):
        wid = lax.axis_index("s") * _NC + lax.axis_index("c")
        base = wid * per_w

        @pl.loop(0, n_chunks)
        def _(j):
            off = base + j * _CHUNK
            pltpu.sync_copy(ids_hbm.at[pl.ds(off, _CHUNK)], idx_v)
            cp_e = pltpu.async_copy(elem_hbm.at[idx_v], erow_v, sem_e)
            cp_h = pltpu.async_copy(hnet_hbm.at[idx_v], hrow_v, sem_h)
            cp_e.wait()
            cp_h.wait()
            pltpu.sync_copy(erow_v, out_e.at[pl.ds(off, _CHUNK)])
            pltpu.sync_copy(hrow_v, out_h.at[pl.ds(off, _CHUNK)])

    return sc_gather(ids_flat, elem_weight, hnet_weight)


def _tc_combine(hnet3, erow3, hrow3, lin_weight, tb=8, interpret=False):
    b, l, nhp = hnet3.shape
    emb = lin_weight.shape[0]

    def body(hnet_ref, e_ref, h_ref, lin_ref, out_ref):
        for t in range(tb):
            scal = lax.dot_general(
                hnet_ref[t], lin_ref[...],
                (((1,), (1,)), ((), ())),
                preferred_element_type=jnp.float32,
            )
            out_ref[t] = e_ref[t, :l, :] + h_ref[t, :l, :] * scal

    lp = erow3.shape[1]
    return pl.pallas_call(
        body,
        grid=(b // tb,),
        in_specs=[
            pl.BlockSpec((tb, l, nhp), lambda i: (i, 0, 0)),
            pl.BlockSpec((tb, lp, emb), lambda i: (i, 0, 0)),
            pl.BlockSpec((tb, lp, emb), lambda i: (i, 0, 0)),
            pl.BlockSpec((emb, nhp), lambda i: (0, 0)),
        ],
        out_specs=pl.BlockSpec((tb, l, emb), lambda i: (i, 0, 0)),
        out_shape=jax.ShapeDtypeStruct((b, l, emb), jnp.float32),
        interpret=interpret,
    )(hnet3, erow3, hrow3, lin_weight)


def kernel(input_ids, hnet_tensor, elem_weight, hnet_weight, lin_weight):
    b, l = input_ids.shape
    emb = elem_weight.shape[1]
    lp = l + (-l) % 8
    n_table = elem_weight.shape[0]
    pad_ids = (jnp.arange(b * (lp - l), dtype=jnp.int32) % n_table).reshape(
        b, lp - l)
    ids_pad = jnp.concatenate([input_ids.astype(jnp.int32), pad_ids], axis=1)
    ids_flat = ids_pad.reshape(b * lp).astype(jnp.int32)
    erow, hrow = _sc_gather_pair(ids_flat, elem_weight, hnet_weight)
    erow3 = erow.reshape(b, lp, emb)
    hrow3 = hrow.reshape(b, lp, emb)
    return _tc_combine(hnet_tensor, erow3, hrow3, lin_weight)


# speedup vs baseline: 3.1103x; 1.1989x over previous
import functools

import jax
import jax.numpy as jnp
from jax import lax
from jax.experimental import pallas as pl
from jax.experimental.pallas import tpu as pltpu
from jax.experimental.pallas import tpu_sc as plsc

_NC = 2
_NS = 16
_NW = _NC * _NS
_CHUNK = 128


def _sc_gather_pair(ids_flat, elem_weight, hnet_weight):
    n = ids_flat.shape[0]
    emb = elem_weight.shape[1]
    per_w = n // _NW
    n_chunks = per_w // _CHUNK
    mesh = plsc.VectorSubcoreMesh(core_axis_name="c", subcore_axis_name="s")

    @functools.partial(
        pl.kernel,
        out_type=(
            jax.ShapeDtypeStruct((n, emb), jnp.float32),
            jax.ShapeDtypeStruct((n, emb), jnp.float32),
        ),
        mesh=mesh,
        scratch_types=[
            pltpu.VMEM((_CHUNK,), jnp.int32),
            pltpu.VMEM((_CHUNK, emb), jnp.float32),
            pltpu.VMEM((_CHUNK, emb), jnp.float32),
            pltpu.SemaphoreType.DMA,
            pltpu.SemaphoreType.DMA,
        ],
        compiler_params=pltpu.CompilerParams(use_tc_tiling_on_sc=True),
    )
    def sc_gather(ids_hbm, elem_hbm, hnet_hbm, out_e, out_h,
                  idx_v, erow_v, hrow_v, sem_e, sem_h):
        wid = lax.axis_index("s") * _NC + lax.axis_index("c")
        base = wid * per_w

        @pl.loop(0, n_chunks)
        def _(j):
            off = base + j * _CHUNK
            pltpu.sync_copy(ids_hbm.at[pl.ds(off, _CHUNK)], idx_v)
            cp_e = pltpu.async_copy(elem_hbm.at[idx_v], erow_v, sem_e)
            cp_h = pltpu.async_copy(hnet_hbm.at[idx_v], hrow_v, sem_h)
            cp_e.wait()
            cp_h.wait()
            pltpu.sync_copy(erow_v, out_e.at[pl.ds(off, _CHUNK)])
            pltpu.sync_copy(hrow_v, out_h.at[pl.ds(off, _CHUNK)])

    return sc_gather(ids_flat, elem_weight, hnet_weight)


def _tc_combine(hnet3, erow3, hrow3, lin_weight, tb=16, interpret=False):
    b, l, nhp = hnet3.shape
    emb = lin_weight.shape[0]

    def body(hnet_ref, e_ref, h_ref, lin_ref, out_ref):
        for t in range(tb):
            scal = lax.dot_general(
                hnet_ref[t], lin_ref[...],
                (((1,), (1,)), ((), ())),
                preferred_element_type=jnp.float32,
            )
            out_ref[t] = e_ref[t, :l, :] + h_ref[t, :l, :] * scal

    lp = erow3.shape[1]
    return pl.pallas_call(
        body,
        grid=(b // tb,),
        in_specs=[
            pl.BlockSpec((tb, l, nhp), lambda i: (i, 0, 0)),
            pl.BlockSpec((tb, lp, emb), lambda i: (i, 0, 0)),
            pl.BlockSpec((tb, lp, emb), lambda i: (i, 0, 0)),
            pl.BlockSpec((emb, nhp), lambda i: (0, 0)),
        ],
        out_specs=pl.BlockSpec((tb, l, emb), lambda i: (i, 0, 0)),
        out_shape=jax.ShapeDtypeStruct((b, l, emb), jnp.float32),
        interpret=interpret,
    )(hnet3, erow3, hrow3, lin_weight)


def kernel(input_ids, hnet_tensor, elem_weight, hnet_weight, lin_weight):
    b, l = input_ids.shape
    emb = elem_weight.shape[1]
    lp = l + (-l) % 8
    n_table = elem_weight.shape[0]
    pad_ids = (jnp.arange(b * (lp - l), dtype=jnp.int32) % n_table).reshape(
        b, lp - l)
    ids_pad = jnp.concatenate([input_ids.astype(jnp.int32), pad_ids], axis=1)
    ids_flat = ids_pad.reshape(b * lp).astype(jnp.int32)
    erow, hrow = _sc_gather_pair(ids_flat, elem_weight, hnet_weight)
    erow3 = erow.reshape(b, lp, emb)
    hrow3 = hrow.reshape(b, lp, emb)
    return _tc_combine(hnet_tensor, erow3, hrow3, lin_weight)


# speedup vs baseline: 3.5454x; 1.1399x over previous
import functools

import jax
import jax.numpy as jnp
from jax import lax
from jax.experimental import pallas as pl
from jax.experimental.pallas import tpu as pltpu
from jax.experimental.pallas import tpu_sc as plsc

_NC = 2
_NS = 16
_NW = _NC * _NS
_CHUNK = 128


def _sc_gather_pair(ids_flat, elem_weight, hnet_weight):
    n = ids_flat.shape[0]
    emb = elem_weight.shape[1]
    per_w = n // _NW
    n_chunks = per_w // _CHUNK
    mesh = plsc.VectorSubcoreMesh(core_axis_name="c", subcore_axis_name="s")

    @functools.partial(
        pl.kernel,
        out_type=(
            jax.ShapeDtypeStruct((n, emb), jnp.float32),
            jax.ShapeDtypeStruct((n, emb), jnp.float32),
        ),
        mesh=mesh,
        scratch_types=[
            pltpu.VMEM((_CHUNK,), jnp.int32),
            pltpu.VMEM((_CHUNK, emb), jnp.float32),
            pltpu.VMEM((_CHUNK, emb), jnp.float32),
            pltpu.SemaphoreType.DMA,
            pltpu.SemaphoreType.DMA,
        ],
        compiler_params=pltpu.CompilerParams(use_tc_tiling_on_sc=True),
    )
    def sc_gather(ids_hbm, elem_hbm, hnet_hbm, out_e, out_h,
                  idx_v, erow_v, hrow_v, sem_e, sem_h):
        wid = lax.axis_index("s") * _NC + lax.axis_index("c")
        base = wid * per_w

        @pl.loop(0, n_chunks)
        def _(j):
            off = base + j * _CHUNK
            pltpu.sync_copy(ids_hbm.at[pl.ds(off, _CHUNK)], idx_v)
            cp_e = pltpu.async_copy(elem_hbm.at[idx_v], erow_v, sem_e)
            cp_h = pltpu.async_copy(hnet_hbm.at[idx_v], hrow_v, sem_h)
            cp_e.wait()
            cp_h.wait()
            pltpu.sync_copy(erow_v, out_e.at[pl.ds(off, _CHUNK)])
            pltpu.sync_copy(hrow_v, out_h.at[pl.ds(off, _CHUNK)])

    return sc_gather(ids_flat, elem_weight, hnet_weight)


def _tc_combine(hnet3, erow3, hrow3, lin_weight, tb=32, interpret=False):
    b, l, nhp = hnet3.shape
    emb = lin_weight.shape[0]

    def body(hnet_ref, e_ref, h_ref, lin_ref, out_ref):
        for t in range(tb):
            scal = lax.dot_general(
                hnet_ref[t], lin_ref[...],
                (((1,), (1,)), ((), ())),
                preferred_element_type=jnp.float32,
            )
            out_ref[t] = e_ref[t, :l, :] + h_ref[t, :l, :] * scal

    lp = erow3.shape[1]
    return pl.pallas_call(
        body,
        grid=(b // tb,),
        in_specs=[
            pl.BlockSpec((tb, l, nhp), lambda i: (i, 0, 0)),
            pl.BlockSpec((tb, lp, emb), lambda i: (i, 0, 0)),
            pl.BlockSpec((tb, lp, emb), lambda i: (i, 0, 0)),
            pl.BlockSpec((emb, nhp), lambda i: (0, 0)),
        ],
        out_specs=pl.BlockSpec((tb, l, emb), lambda i: (i, 0, 0)),
        out_shape=jax.ShapeDtypeStruct((b, l, emb), jnp.float32),
        interpret=interpret,
    )(hnet3, erow3, hrow3, lin_weight)


def kernel(input_ids, hnet_tensor, elem_weight, hnet_weight, lin_weight):
    b, l = input_ids.shape
    emb = elem_weight.shape[1]
    lp = l + (-l) % 8
    n_table = elem_weight.shape[0]
    pad_ids = (jnp.arange(b * (lp - l), dtype=jnp.int32) % n_table).reshape(
        b, lp - l)
    ids_pad = jnp.concatenate([input_ids.astype(jnp.int32), pad_ids], axis=1)
    ids_flat = ids_pad.reshape(b * lp).astype(jnp.int32)
    erow, hrow = _sc_gather_pair(ids_flat, elem_weight, hnet_weight)
    erow3 = erow.reshape(b, lp, emb)
    hrow3 = hrow.reshape(b, lp, emb)
    return _tc_combine(hnet_tensor, erow3, hrow3, lin_weight)


# speedup vs baseline: 3.7748x; 1.0647x over previous
import functools

import jax
import jax.numpy as jnp
from jax import lax
from jax.experimental import pallas as pl
from jax.experimental.pallas import tpu as pltpu
from jax.experimental.pallas import tpu_sc as plsc

_NC = 2
_NS = 16
_NW = _NC * _NS
_CHUNK = 128


def _sc_gather_pair(ids_flat, elem_weight, hnet_weight):
    n = ids_flat.shape[0]
    emb = elem_weight.shape[1]
    per_w = n // _NW
    n_chunks = per_w // _CHUNK
    mesh = plsc.VectorSubcoreMesh(core_axis_name="c", subcore_axis_name="s")

    @functools.partial(
        pl.kernel,
        out_type=(
            jax.ShapeDtypeStruct((n, emb), jnp.float32),
            jax.ShapeDtypeStruct((n, emb), jnp.float32),
        ),
        mesh=mesh,
        scratch_types=[
            pltpu.VMEM((_CHUNK,), jnp.int32),
            pltpu.VMEM((_CHUNK, emb), jnp.float32),
            pltpu.VMEM((_CHUNK, emb), jnp.float32),
            pltpu.SemaphoreType.DMA,
            pltpu.SemaphoreType.DMA,
        ],
        compiler_params=pltpu.CompilerParams(use_tc_tiling_on_sc=True),
    )
    def sc_gather(ids_hbm, elem_hbm, hnet_hbm, out_e, out_h,
                  idx_v, erow_v, hrow_v, sem_e, sem_h):
        wid = lax.axis_index("s") * _NC + lax.axis_index("c")
        base = wid * per_w

        @pl.loop(0, n_chunks)
        def _(j):
            off = base + j * _CHUNK
            pltpu.sync_copy(ids_hbm.at[pl.ds(off, _CHUNK)], idx_v)
            cp_e = pltpu.async_copy(elem_hbm.at[idx_v], erow_v, sem_e)
            cp_h = pltpu.async_copy(hnet_hbm.at[idx_v], hrow_v, sem_h)
            cp_e.wait()
            cp_h.wait()
            pltpu.sync_copy(erow_v, out_e.at[pl.ds(off, _CHUNK)])
            pltpu.sync_copy(hrow_v, out_h.at[pl.ds(off, _CHUNK)])

    return sc_gather(ids_flat, elem_weight, hnet_weight)


def _tc_combine(hnet3, erow3, hrow3, lin_weight, tb=64, interpret=False):
    b, l, nhp = hnet3.shape
    emb = lin_weight.shape[0]

    def body(hnet_ref, e_ref, h_ref, lin_ref, out_ref):
        for t in range(tb):
            scal = lax.dot_general(
                hnet_ref[t], lin_ref[...],
                (((1,), (1,)), ((), ())),
                preferred_element_type=jnp.float32,
            )
            out_ref[t] = e_ref[t, :l, :] + h_ref[t, :l, :] * scal

    lp = erow3.shape[1]
    return pl.pallas_call(
        body,
        grid=(b // tb,),
        in_specs=[
            pl.BlockSpec((tb, l, nhp), lambda i: (i, 0, 0)),
            pl.BlockSpec((tb, lp, emb), lambda i: (i, 0, 0)),
            pl.BlockSpec((tb, lp, emb), lambda i: (i, 0, 0)),
            pl.BlockSpec((emb, nhp), lambda i: (0, 0)),
        ],
        out_specs=pl.BlockSpec((tb, l, emb), lambda i: (i, 0, 0)),
        out_shape=jax.ShapeDtypeStruct((b, l, emb), jnp.float32),
        interpret=interpret,
    )(hnet3, erow3, hrow3, lin_weight)


def kernel(input_ids, hnet_tensor, elem_weight, hnet_weight, lin_weight):
    b, l = input_ids.shape
    emb = elem_weight.shape[1]
    lp = l + (-l) % 8
    n_table = elem_weight.shape[0]
    pad_ids = (jnp.arange(b * (lp - l), dtype=jnp.int32) % n_table).reshape(
        b, lp - l)
    ids_pad = jnp.concatenate([input_ids.astype(jnp.int32), pad_ids], axis=1)
    ids_flat = ids_pad.reshape(b * lp).astype(jnp.int32)
    erow, hrow = _sc_gather_pair(ids_flat, elem_weight, hnet_weight)
    erow3 = erow.reshape(b, lp, emb)
    hrow3 = hrow.reshape(b, lp, emb)
    return _tc_combine(hnet_tensor, erow3, hrow3, lin_weight)


# speedup vs baseline: 3.8349x; 1.0159x over previous
import functools

import jax
import jax.numpy as jnp
from jax import lax
from jax.experimental import pallas as pl
from jax.experimental.pallas import tpu as pltpu
from jax.experimental.pallas import tpu_sc as plsc

_NC = 2
_NS = 16
_NW = _NC * _NS
_CHUNK = 128


def _sc_gather_pair(ids_flat, elem_weight, hnet_weight):
    n = ids_flat.shape[0]
    emb = elem_weight.shape[1]
    per_w = n // _NW
    n_chunks = per_w // _CHUNK
    mesh = plsc.VectorSubcoreMesh(core_axis_name="c", subcore_axis_name="s")

    @functools.partial(
        pl.kernel,
        out_type=(
            jax.ShapeDtypeStruct((n, emb), jnp.float32),
            jax.ShapeDtypeStruct((n, emb), jnp.float32),
        ),
        mesh=mesh,
        scratch_types=[
            pltpu.VMEM((_CHUNK,), jnp.int32),
            pltpu.VMEM((_CHUNK, emb), jnp.float32),
            pltpu.VMEM((_CHUNK, emb), jnp.float32),
            pltpu.SemaphoreType.DMA,
            pltpu.SemaphoreType.DMA,
        ],
        compiler_params=pltpu.CompilerParams(use_tc_tiling_on_sc=True),
    )
    def sc_gather(ids_hbm, elem_hbm, hnet_hbm, out_e, out_h,
                  idx_v, erow_v, hrow_v, sem_e, sem_h):
        wid = lax.axis_index("s") * _NC + lax.axis_index("c")
        base = wid * per_w

        @pl.loop(0, n_chunks)
        def _(j):
            off = base + j * _CHUNK
            pltpu.sync_copy(ids_hbm.at[pl.ds(off, _CHUNK)], idx_v)
            cp_e = pltpu.async_copy(elem_hbm.at[idx_v], erow_v, sem_e)
            cp_h = pltpu.async_copy(hnet_hbm.at[idx_v], hrow_v, sem_h)
            cp_e.wait()
            cp_h.wait()
            pltpu.sync_copy(erow_v, out_e.at[pl.ds(off, _CHUNK)])
            pltpu.sync_copy(hrow_v, out_h.at[pl.ds(off, _CHUNK)])

    return sc_gather(ids_flat, elem_weight, hnet_weight)


def _tc_combine(hnet3, erow3, hrow3, lin_weight, tb=128, interpret=False):
    b, l, nhp = hnet3.shape
    emb = lin_weight.shape[0]

    def body(hnet_ref, e_ref, h_ref, lin_ref, out_ref):
        for t in range(tb):
            scal = lax.dot_general(
                hnet_ref[t], lin_ref[...],
                (((1,), (1,)), ((), ())),
                preferred_element_type=jnp.float32,
            )
            out_ref[t] = e_ref[t, :l, :] + h_ref[t, :l, :] * scal

    lp = erow3.shape[1]
    return pl.pallas_call(
        body,
        grid=(b // tb,),
        in_specs=[
            pl.BlockSpec((tb, l, nhp), lambda i: (i, 0, 0)),
            pl.BlockSpec((tb, lp, emb), lambda i: (i, 0, 0)),
            pl.BlockSpec((tb, lp, emb), lambda i: (i, 0, 0)),
            pl.BlockSpec((emb, nhp), lambda i: (0, 0)),
        ],
        out_specs=pl.BlockSpec((tb, l, emb), lambda i: (i, 0, 0)),
        out_shape=jax.ShapeDtypeStruct((b, l, emb), jnp.float32),
        interpret=interpret,
    )(hnet3, erow3, hrow3, lin_weight)


def kernel(input_ids, hnet_tensor, elem_weight, hnet_weight, lin_weight):
    b, l = input_ids.shape
    emb = elem_weight.shape[1]
    lp = l + (-l) % 8
    n_table = elem_weight.shape[0]
    pad_ids = (jnp.arange(b * (lp - l), dtype=jnp.int32) % n_table).reshape(
        b, lp - l)
    ids_pad = jnp.concatenate([input_ids.astype(jnp.int32), pad_ids], axis=1)
    ids_flat = ids_pad.reshape(b * lp).astype(jnp.int32)
    erow, hrow = _sc_gather_pair(ids_flat, elem_weight, hnet_weight)
    erow3 = erow.reshape(b, lp, emb)
    hrow3 = hrow.reshape(b, lp, emb)
    return _tc_combine(hnet_tensor, erow3, hrow3, lin_weight)


# speedup vs baseline: 4.1091x; 1.0715x over previous
import functools

import jax
import jax.numpy as jnp
from jax import lax
from jax.experimental import pallas as pl
from jax.experimental.pallas import tpu as pltpu
from jax.experimental.pallas import tpu_sc as plsc

_NC = 2
_NS = 16
_NW = _NC * _NS
_CHUNK = 112
_DEPTH = 4


def _sc_gather_pair(ids_flat, elem_weight, hnet_weight):
    n = ids_flat.shape[0]
    emb = elem_weight.shape[1]
    per_w = n // _NW
    n_chunks = per_w // _CHUNK
    assert n_chunks % _DEPTH == 0
    mesh = plsc.VectorSubcoreMesh(core_axis_name="c", subcore_axis_name="s")

    buf_types = []
    for _ in range(_DEPTH):
        buf_types.append(pltpu.VMEM((_CHUNK, emb), jnp.float32))
        buf_types.append(pltpu.VMEM((_CHUNK, emb), jnp.float32))
        buf_types.append(pltpu.SemaphoreType.DMA)
        buf_types.append(pltpu.SemaphoreType.DMA)

    @functools.partial(
        pl.kernel,
        out_type=(
            jax.ShapeDtypeStruct((n, emb), jnp.float32),
            jax.ShapeDtypeStruct((n, emb), jnp.float32),
        ),
        mesh=mesh,
        scratch_types=[pltpu.VMEM((per_w,), jnp.int32)] + buf_types,
        compiler_params=pltpu.CompilerParams(use_tc_tiling_on_sc=True),
    )
    def sc_gather(ids_hbm, elem_hbm, hnet_hbm, out_e, out_h, idx_all, *bufs):
        ebuf = [bufs[4 * s] for s in range(_DEPTH)]
        hbuf = [bufs[4 * s + 1] for s in range(_DEPTH)]
        sem_e = [bufs[4 * s + 2] for s in range(_DEPTH)]
        sem_h = [bufs[4 * s + 3] for s in range(_DEPTH)]

        wid = lax.axis_index("s") * _NC + lax.axis_index("c")
        base = wid * per_w
        pltpu.sync_copy(ids_hbm.at[pl.ds(base, per_w)], idx_all)

        def fire(s, chunk):
            isl = idx_all.at[pl.ds(chunk * _CHUNK, _CHUNK)]
            pltpu.make_async_copy(elem_hbm.at[isl], ebuf[s], sem_e[s]).start()
            pltpu.make_async_copy(hnet_hbm.at[isl], hbuf[s], sem_h[s]).start()

        def drain(s, chunk):
            isl = idx_all.at[pl.ds(chunk * _CHUNK, _CHUNK)]
            pltpu.make_async_copy(elem_hbm.at[isl], ebuf[s], sem_e[s]).wait()
            pltpu.make_async_copy(hnet_hbm.at[isl], hbuf[s], sem_h[s]).wait()
            off = base + chunk * _CHUNK
            pltpu.sync_copy(ebuf[s], out_e.at[pl.ds(off, _CHUNK)])
            pltpu.sync_copy(hbuf[s], out_h.at[pl.ds(off, _CHUNK)])

        for s in range(_DEPTH):
            fire(s, s)

        @pl.loop(0, n_chunks // _DEPTH - 1)
        def _(m):
            for s in range(_DEPTH):
                cur = m * _DEPTH + s
                drain(s, cur)
                fire(s, cur + _DEPTH)

        last = n_chunks - _DEPTH
        for s in range(_DEPTH):
            drain(s, last + s)

    return sc_gather(ids_flat, elem_weight, hnet_weight)


def _tc_combine(hnet3, erow3, hrow3, lin_weight, tb=128, interpret=False):
    b, l, nhp = hnet3.shape
    emb = lin_weight.shape[0]

    def body(hnet_ref, e_ref, h_ref, lin_ref, out_ref):
        for t in range(tb):
            scal = lax.dot_general(
                hnet_ref[t], lin_ref[...],
                (((1,), (1,)), ((), ())),
                preferred_element_type=jnp.float32,
            )
            out_ref[t] = e_ref[t, :l, :] + h_ref[t, :l, :] * scal

    lp = erow3.shape[1]
    return pl.pallas_call(
        body,
        grid=(b // tb,),
        in_specs=[
            pl.BlockSpec((tb, l, nhp), lambda i: (i, 0, 0)),
            pl.BlockSpec((tb, lp, emb), lambda i: (i, 0, 0)),
            pl.BlockSpec((tb, lp, emb), lambda i: (i, 0, 0)),
            pl.BlockSpec((emb, nhp), lambda i: (0, 0)),
        ],
        out_specs=pl.BlockSpec((tb, l, emb), lambda i: (i, 0, 0)),
        out_shape=jax.ShapeDtypeStruct((b, l, emb), jnp.float32),
        interpret=interpret,
    )(hnet3, erow3, hrow3, lin_weight)


def kernel(input_ids, hnet_tensor, elem_weight, hnet_weight, lin_weight):
    b, l = input_ids.shape
    emb = elem_weight.shape[1]
    lp = l + (-l) % 8
    n_table = elem_weight.shape[0]
    pad_ids = (jnp.arange(b * (lp - l), dtype=jnp.int32) % n_table).reshape(
        b, lp - l)
    ids_pad = jnp.concatenate([input_ids.astype(jnp.int32), pad_ids], axis=1)
    ids_flat = ids_pad.reshape(b * lp).astype(jnp.int32)
    erow, hrow = _sc_gather_pair(ids_flat, elem_weight, hnet_weight)
    erow3 = erow.reshape(b, lp, emb)
    hrow3 = hrow.reshape(b, lp, emb)
    return _tc_combine(hnet_tensor, erow3, hrow3, lin_weight)
